# Initial kernel scaffold; baseline (speedup 1.0000x reference)
import jax
import jax.numpy as jnp
from jax.experimental import pallas as pl


def kernel(x_torso, x_joint, edge_index_tj, edge_index_jt, edge_index_jj, Wi_t, bi_t, Wi_j, bi_j, Wr1_tj, br1_tj, Wq1_tj, Wr1_jt, br1_jt, Wq1_jt, Wr1_jj, br1_jj, Wq1_jj, Wr2_tj, br2_tj, Wq2_tj, Wr2_jt, br2_jt, Wq2_jt, Wr2_jj, br2_jj, Wq2_jj, Wo_t, bo_t, Wo_j, bo_j):
    raise NotImplementedError("write your pallas kernel here")


# speedup vs baseline: 4.0287x; 4.0287x over previous
import functools
import math

import jax
import jax.numpy as jnp
from jax import lax
from jax.experimental import pallas as pl
from jax.experimental.pallas import tpu as pltpu
from jax.experimental.pallas import tpu_sc as plsc

N = 50000
E = 200000
D = 128
H = 64
HW = 32
O = 16
_BIAS = math.log(math.exp(1.0) - 1.0)

NC = 2
NS = 16
PT = 3128
PT_LAST = N - (NS - 1) * PT
ACC_ROWS = NS * PT
CH = 112
E_PAD = 200704
NCH = E_PAD // CH
CPT = NCH // NS
HSLAB = CPT // 2
RING = 4


def _sc_segsum_body(gjt_lo, gjt_hi, gtj_lo, gtj_hi, gjj_lo, gjj_hi,
                    s_jt, d_jt, s_tj, d_tj, s_jj, d_jj,
                    mt_lo, mt_hi, mj_lo, mj_hi,
                    acc, srcb, dstb, rows, gs, ss):
    c = lax.axis_index("c")
    s = lax.axis_index("s")
    c_lo = s * CPT

    zv = jnp.zeros((16,), jnp.float32)

    def zero_rows0():
        def _zb(i, carry):
            for l in range(HW // 16):
                rows[0, i, pl.ds(l * 16, 16)] = zv
            return carry
        lax.fori_loop(0, CH, _zb, 0)

    def zero_acc():
        r0 = s * PT
        nfull = PT // CH
        for i in range(nfull):
            pltpu.sync_copy(rows.at[0], acc.at[pl.ds(r0 + i * CH, CH)])
        rem = PT - nfull * CH
        if rem:
            pltpu.sync_copy(rows.at[0, pl.ds(0, rem)],
                            acc.at[pl.ds(r0 + nfull * CH, rem)])

    def accumulate(gsrc, s2d, d2d):
        def fire_g(k, p):
            pltpu.async_copy(gsrc.at[srcb.at[k]], rows.at[p], gs.at[p])

        def drain_g(k, p):
            pltpu.make_async_copy(gsrc.at[srcb.at[k]], rows.at[p], gs.at[p]).wait()

        def fire_s(k, p):
            pltpu.async_copy(rows.at[p], acc.at[dstb.at[k]], ss.at[p], add=True)

        def drain_s(k, p):
            pltpu.make_async_copy(rows.at[p], acc.at[dstb.at[k]], ss.at[p]).wait()

        def body(k4, carry):
            for p in range(RING):
                k = RING * k4 + p
                drain_g(k, p)
                fire_s(k, p)

                @pl.when(k + RING < HSLAB)
                def _():
                    drain_s(k, p)
                    fire_g(k + RING, p)

            return carry

        for h in range(CPT // HSLAB):
            pltpu.sync_copy(s2d.at[pl.ds(c_lo + h * HSLAB, HSLAB)], srcb)
            pltpu.sync_copy(d2d.at[pl.ds(c_lo + h * HSLAB, HSLAB)], dstb)
            for p in range(RING):
                fire_g(p, p)
            lax.fori_loop(0, HSLAB // RING, body, 0)
            for p in range(RING):
                drain_s(HSLAB - RING + p, p)

    def dump(out_hbm):
        r0 = s * PT

        @pl.when(s < NS - 1)
        def _():
            pltpu.sync_copy(acc.at[pl.ds(r0, PT)], out_hbm.at[pl.ds(r0, PT)])

        @pl.when(s == NS - 1)
        def _():
            pltpu.sync_copy(acc.at[pl.ds(r0, PT_LAST)],
                            out_hbm.at[pl.ds(r0, PT_LAST)])

    def conv_pass(g_lo, g_hi, s2d, d2d):
        @pl.when(c == 0)
        def _():
            accumulate(g_lo, s2d, d2d)

        @pl.when(c == 1)
        def _():
            accumulate(g_hi, s2d, d2d)

    def dump_pass(out_lo, out_hi):
        @pl.when(c == 0)
        def _():
            dump(out_lo)

        @pl.when(c == 1)
        def _():
            dump(out_hi)

    zero_rows0()
    zero_acc()
    plsc.subcore_barrier()
    conv_pass(gjt_lo, gjt_hi, s_jt, d_jt)
    plsc.subcore_barrier()
    dump_pass(mt_lo, mt_hi)
    zero_rows0()
    zero_acc()
    plsc.subcore_barrier()
    conv_pass(gtj_lo, gtj_hi, s_tj, d_tj)
    conv_pass(gjj_lo, gjj_hi, s_jj, d_jj)
    plsc.subcore_barrier()
    dump_pass(mj_lo, mj_hi)


_sc_segsum = pl.kernel(
    _sc_segsum_body,
    out_type=tuple(jax.ShapeDtypeStruct((N, HW), jnp.float32) for _ in range(4)),
    mesh=plsc.VectorSubcoreMesh(core_axis_name="c", subcore_axis_name="s"),
    scratch_types=(
        pltpu.VMEM_SHARED((ACC_ROWS, HW), jnp.float32),
        pltpu.VMEM((HSLAB, CH), jnp.int32),
        pltpu.VMEM((HSLAB, CH), jnp.int32),
        pltpu.VMEM((RING, CH, HW), jnp.float32),
        pltpu.SemaphoreType.DMA((RING,)),
        pltpu.SemaphoreType.DMA((RING,)),
    ),
    compiler_params=pltpu.CompilerParams(use_tc_tiling_on_sc=False),
)


R = 2000
_P = jax.lax.Precision.DEFAULT


def _dot(a, b):
    return jnp.dot(a, b, precision=_P, preferred_element_type=jnp.float32)


def _f1_t_body(x, wi, bi, wr_lo, wr_hi, wq, bq, glo, ghi, q_out):
    h = _dot(x[...], wi[...]) + bi[...]
    glo[...] = _dot(h, wr_lo[...])
    ghi[...] = _dot(h, wr_hi[...])
    q_out[...] = _dot(h, wq[...]) + bq[...]


def _f1_j_body(x, wi, bi, wa_lo, wa_hi, wb_lo, wb_hi, wq, bq,
               galo, gahi, gblo, gbhi, q_out):
    h = _dot(x[...], wi[...]) + bi[...]
    galo[...] = _dot(h, wa_lo[...])
    gahi[...] = _dot(h, wa_hi[...])
    gblo[...] = _dot(h, wb_lo[...])
    gbhi[...] = _dot(h, wb_hi[...])
    q_out[...] = _dot(h, wq[...]) + bq[...]


def _f2_t_body(mlo, mhi, q_in, wr_lo, wr_hi, wq, bq, glo, ghi, q_out):
    msg = jnp.concatenate([mlo[...], mhi[...]], axis=1)
    h = jnp.tanh(msg + q_in[...])
    glo[...] = _dot(h, wr_lo[...])
    ghi[...] = _dot(h, wr_hi[...])
    q_out[...] = _dot(h, wq[...]) + bq[...]


def _f2_j_body(mlo, mhi, q_in, wa_lo, wa_hi, wb_lo, wb_hi, wq, bq,
               galo, gahi, gblo, gbhi, q_out):
    msg = jnp.concatenate([mlo[...], mhi[...]], axis=1)
    h = jnp.tanh(msg + q_in[...])
    galo[...] = _dot(h, wa_lo[...])
    gahi[...] = _dot(h, wa_hi[...])
    gblo[...] = _dot(h, wb_lo[...])
    gbhi[...] = _dot(h, wb_hi[...])
    q_out[...] = _dot(h, wq[...]) + bq[...]


def _f3_body(mlo, mhi, q_in, wo, bo, y_out, z_out):
    msg = jnp.concatenate([mlo[...], mhi[...]], axis=1)
    h = jnp.tanh(msg + q_in[...])
    y = jnp.tanh(_dot(h, wo[...]) + bo[...])
    y_out[...] = y
    v = y + _BIAS
    sp = jnp.log1p(jnp.exp(-jnp.abs(v))) + jnp.maximum(v, 0.0)
    z_out[...] = jnp.maximum(sp, 1e-4)


def _row_spec(cols):
    return pl.BlockSpec((R, cols), lambda i: (i, 0))


def _w_spec(r, c):
    return pl.BlockSpec((r, c), lambda i: (0, 0))


def _call(body, in_cols, w_shapes, out_cols):
    grid = N // R
    in_specs = [_row_spec(cc) for cc in in_cols] + [
        _w_spec(*sh) for sh in w_shapes]
    return pl.pallas_call(
        body,
        grid=(grid,),
        in_specs=in_specs,
        out_specs=[_row_spec(cc) for cc in out_cols],
        out_shape=[jax.ShapeDtypeStruct((N, cc), jnp.float32) for cc in out_cols],
    )


_f1_t = _call(_f1_t_body, [D],
              [(D, H), (1, H), (H, HW), (H, HW), (H, H), (1, H)],
              [HW, HW, H])
_f1_j = _call(_f1_j_body, [D],
              [(D, H), (1, H), (H, HW), (H, HW), (H, HW), (H, HW), (H, H), (1, H)],
              [HW, HW, HW, HW, H])
_f2_t = _call(_f2_t_body, [HW, HW, H],
              [(H, HW), (H, HW), (H, H), (1, H)],
              [HW, HW, H])
_f2_j = _call(_f2_j_body, [HW, HW, H],
              [(H, HW), (H, HW), (H, HW), (H, HW), (H, H), (1, H)],
              [HW, HW, HW, HW, H])
_f3 = _call(_f3_body, [HW, HW, H], [(H, O), (1, O)], [O, O])


def kernel(x_torso, x_joint, edge_index_tj, edge_index_jt, edge_index_jj,
           Wi_t, bi_t, Wi_j, bi_j,
           Wr1_tj, br1_tj, Wq1_tj, Wr1_jt, br1_jt, Wq1_jt, Wr1_jj, br1_jj, Wq1_jj,
           Wr2_tj, br2_tj, Wq2_tj, Wr2_jt, br2_jt, Wq2_jt, Wr2_jj, br2_jj, Wq2_jj,
           Wo_t, bo_t, Wo_j, bo_j):
    pad_src = (jnp.arange(E_PAD - E, dtype=jnp.int32) * 41) % N
    pad_dst = N + (jnp.arange(E_PAD - E, dtype=jnp.int32) & 7)

    def _prep(ei):
        return (jnp.concatenate([ei[0], pad_src]).reshape(NCH, CH),
                jnp.concatenate([ei[1], pad_dst]).reshape(NCH, CH))

    s_tj, d_tj = _prep(edge_index_tj)
    s_jt, d_jt = _prep(edge_index_jt)
    s_jj, d_jj = _prep(edge_index_jj)
    r2 = lambda b: b.reshape(1, -1)
    lo = lambda w: w[:, :HW]
    hi = lambda w: w[:, HW:]
    wq1_j = Wq1_tj + Wq1_jj
    bq1_j = r2(br1_tj + br1_jj)
    wq2_j = Wq2_tj + Wq2_jj
    bq2_j = r2(br2_tj + br2_jj)

    gtj_lo, gtj_hi, q_t = _f1_t(x_torso, Wi_t, r2(bi_t),
                                lo(Wr1_tj), hi(Wr1_tj), Wq1_jt, r2(br1_jt))
    gjt_lo, gjt_hi, gjj_lo, gjj_hi, q_j = _f1_j(
        x_joint, Wi_j, r2(bi_j), lo(Wr1_jt), hi(Wr1_jt),
        lo(Wr1_jj), hi(Wr1_jj), wq1_j, bq1_j)
    mt_lo, mt_hi, mj_lo, mj_hi = _sc_segsum(
        gjt_lo, gjt_hi, gtj_lo, gtj_hi, gjj_lo, gjj_hi,
        s_jt, d_jt, s_tj, d_tj, s_jj, d_jj)
    gtj_lo, gtj_hi, q_t = _f2_t(mt_lo, mt_hi, q_t,
                                lo(Wr2_tj), hi(Wr2_tj), Wq2_jt, r2(br2_jt))
    gjt_lo, gjt_hi, gjj_lo, gjj_hi, q_j = _f2_j(
        mj_lo, mj_hi, q_j, lo(Wr2_jt), hi(Wr2_jt),
        lo(Wr2_jj), hi(Wr2_jj), wq2_j, bq2_j)
    mt_lo, mt_hi, mj_lo, mj_hi = _sc_segsum(
        gjt_lo, gjt_hi, gtj_lo, gtj_hi, gjj_lo, gjj_hi,
        s_jt, d_jt, s_tj, d_tj, s_jj, d_jj)
    y_t, z_t = _f3(mt_lo, mt_hi, q_t, Wo_t, r2(bo_t))
    y_j, z_j = _f3(mj_lo, mj_hi, q_j, Wo_j, r2(bo_j))
    half = O // 2
    return (y_t[:, :half], z_t[:, half:], y_j[:, :half], z_j[:, half:])


# speedup vs baseline: 7.1262x; 1.7689x over previous
import math

import jax
import jax.numpy as jnp
from jax import lax
from jax.experimental import pallas as pl
from jax.experimental.pallas import tpu as pltpu
from jax.experimental.pallas import tpu_sc as plsc

N = 50000
E = 200000
D = 128
H = 64
HW = 32
O = 16
_BIAS = math.log(math.exp(1.0) - 1.0)

NC = 2
NS = 16
PT = 3128
PT_LAST = N - (NS - 1) * PT
ACC_ROWS = NS * PT
CH = 112
E_PAD = 200704
NCH = E_PAD // CH
CPT = NCH // NS
HSLAB = CPT // 2
RING = 4


def _sc_segsum_body(va, vb, q128,
                    sl_jt, sh_jt, d_jt, sl_tj, sh_tj, d_tj,
                    sl_jj, sh_jj, d_jj,
                    mt128, mj128,
                    acc, srcb, dstb, rows, gs, ss):
    c = lax.axis_index("c")
    s = lax.axis_index("s")
    c_lo = s * CPT

    def init_acc(col0):
        r0 = s * PT

        @pl.when(s < NS - 1)
        def _():
            pltpu.sync_copy(q128.at[pl.ds(r0, PT), pl.ds(col0, HW)],
                            acc.at[pl.ds(r0, PT)])

        @pl.when(s == NS - 1)
        def _():
            pltpu.sync_copy(q128.at[pl.ds(r0, PT_LAST), pl.ds(col0, HW)],
                            acc.at[pl.ds(r0, PT_LAST)])
            pltpu.sync_copy(q128.at[pl.ds(0, PT - PT_LAST), pl.ds(col0, HW)],
                            acc.at[pl.ds(r0 + PT_LAST, PT - PT_LAST)])

    def init_pass(base):
        @pl.when(c == 0)
        def _():
            init_acc(base)

        @pl.when(c == 1)
        def _():
            init_acc(base + HW)

    def accumulate(gsrc, s2d, d2d):
        def fire_g(k, p):
            pltpu.async_copy(gsrc.at[srcb.at[k]], rows.at[p], gs.at[p])

        def drain_g(k, p):
            pltpu.make_async_copy(gsrc.at[srcb.at[k]], rows.at[p], gs.at[p]).wait()

        def fire_s(k, p):
            pltpu.async_copy(rows.at[p], acc.at[dstb.at[k]], ss.at[p], add=True)

        def drain_s(k, p):
            pltpu.make_async_copy(rows.at[p], acc.at[dstb.at[k]], ss.at[p]).wait()

        def body(k4, carry):
            for p in range(RING):
                k = RING * k4 + p
                drain_g(k, p)
                fire_s(k, p)

                @pl.when(k + RING < HSLAB)
                def _():
                    drain_s(k, p)
                    fire_g(k + RING, p)

            return carry

        for h in range(CPT // HSLAB):
            pltpu.sync_copy(s2d.at[pl.ds(c_lo + h * HSLAB, HSLAB)], srcb)
            pltpu.sync_copy(d2d.at[pl.ds(c_lo + h * HSLAB, HSLAB)], dstb)
            for p in range(RING):
                fire_g(p, p)
            lax.fori_loop(0, HSLAB // RING, body, 0)
            for p in range(RING):
                drain_s(HSLAB - RING + p, p)

    def dump(out128):
        r0 = s * PT

        def to(col0, n):
            pltpu.sync_copy(acc.at[pl.ds(r0, n)],
                            out128.at[pl.ds(r0, n), pl.ds(col0, HW)])

        @pl.when((c == 0) & (s < NS - 1))
        def _():
            to(0, PT)

        @pl.when((c == 0) & (s == NS - 1))
        def _():
            to(0, PT_LAST)

        @pl.when((c == 1) & (s < NS - 1))
        def _():
            to(HW, PT)

        @pl.when((c == 1) & (s == NS - 1))
        def _():
            to(HW, PT_LAST)

    def conv_pass(gview, s_lo, s_hi, d2d):
        @pl.when(c == 0)
        def _():
            accumulate(gview, s_lo, d2d)

        @pl.when(c == 1)
        def _():
            accumulate(gview, s_hi, d2d)

    init_pass(0)
    plsc.subcore_barrier()
    conv_pass(va, sl_jt, sh_jt, d_jt)
    plsc.subcore_barrier()
    dump(mt128)
    init_pass(2 * HW)
    plsc.subcore_barrier()
    conv_pass(va, sl_tj, sh_tj, d_tj)
    conv_pass(vb, sl_jj, sh_jj, d_jj)
    plsc.subcore_barrier()
    dump(mj128)


_sc_segsum = pl.kernel(
    _sc_segsum_body,
    out_type=tuple(jax.ShapeDtypeStruct((N, 4 * HW), jnp.float32)
                   for _ in range(2)),
    mesh=plsc.VectorSubcoreMesh(core_axis_name="c", subcore_axis_name="s"),
    scratch_types=(
        pltpu.VMEM_SHARED((ACC_ROWS, HW), jnp.float32),
        pltpu.VMEM((HSLAB, CH), jnp.int32),
        pltpu.VMEM((HSLAB, CH), jnp.int32),
        pltpu.VMEM((RING, CH, HW), jnp.float32),
        pltpu.SemaphoreType.DMA((RING,)),
        pltpu.SemaphoreType.DMA((RING,)),
    ),
    compiler_params=pltpu.CompilerParams(use_tc_tiling_on_sc=False),
)


R = 2000
_P = jax.lax.Precision.DEFAULT


def _dot(a, b):
    return jnp.dot(a, b, precision=_P, preferred_element_type=jnp.float32)


def _cat(a, b):
    return jnp.concatenate([a, b], axis=1)


def _f1_body(xt, xj, wit, bit, wij, bij, wr_tj, wr_jt, wr_jj,
             wq_t, bq_t, wq_j, bq_j, a_out, b_out, q_out):
    ht = _dot(xt[...], wit[...]) + bit[...]
    hj = _dot(xj[...], wij[...]) + bij[...]
    a_out[...] = _cat(_dot(ht, wr_tj[...]), _dot(hj, wr_jt[...]))
    b_out[...] = _cat(_dot(hj, wr_jj[...]), jnp.zeros((R, H), jnp.float32))
    q_out[...] = _cat(_dot(ht, wq_t[...]) + bq_t[...],
                      _dot(hj, wq_j[...]) + bq_j[...])


def _f2_body(mt, mj, wr_tj, wr_jt, wr_jj, wq_t, bq_t, wq_j, bq_j,
             a_out, b_out, q_out):
    ht = jnp.tanh(mt[:, :H])
    hj = jnp.tanh(mj[:, :H])
    a_out[...] = _cat(_dot(ht, wr_tj[...]), _dot(hj, wr_jt[...]))
    b_out[...] = _cat(_dot(hj, wr_jj[...]), jnp.zeros((R, H), jnp.float32))
    q_out[...] = _cat(_dot(ht, wq_t[...]) + bq_t[...],
                      _dot(hj, wq_j[...]) + bq_j[...])


def _f3_body(mt, mj, wot, bot, woj, boj,
             loc_t_out, scale_t_out, loc_j_out, scale_j_out):
    def head(m, wo, bo, loc_ref, scale_ref):
        h = jnp.tanh(m[:, :H])
        y = jnp.tanh(_dot(h, wo[...]) + bo[...])
        loc_ref[...] = y[:, :O // 2]
        v = y[:, O // 2:] + _BIAS
        sp = jnp.log1p(jnp.exp(-jnp.abs(v))) + jnp.maximum(v, 0.0)
        scale_ref[...] = jnp.maximum(sp, 1e-4)

    head(mt, wot, bot, loc_t_out, scale_t_out)
    head(mj, woj, boj, loc_j_out, scale_j_out)


def _spec(rows, cols):
    return pl.BlockSpec((rows, cols), lambda i: (i, 0))


def _w_spec(r, cc):
    return pl.BlockSpec((r, cc), lambda i: (0, 0))


def _call(body, in_rc, w_shapes, out_rc):
    grid = N // R
    in_specs = [_spec(*rc) for rc in in_rc] + [_w_spec(*sh) for sh in w_shapes]
    return pl.pallas_call(
        body,
        grid=(grid,),
        in_specs=in_specs,
        out_specs=[_spec(*rc) for rc in out_rc],
        out_shape=[jax.ShapeDtypeStruct((N, rc[1]), jnp.float32)
                   for rc in out_rc],
    )


_P128 = (R, 4 * HW)
_WH = (H, H)
_B = (1, H)
_f1 = _call(_f1_body, [(R, D), (R, D)],
            [(D, H), _B, (D, H), _B, _WH, _WH, _WH, _WH, _B, _WH, _B],
            [_P128, _P128, _P128])
_f2 = _call(_f2_body, [_P128, _P128],
            [_WH, _WH, _WH, _WH, _B, _WH, _B],
            [_P128, _P128, _P128])
_f3 = _call(_f3_body, [_P128, _P128],
            [(H, O), (1, O), (H, O), (1, O)],
            [(R, O // 2)] * 4)


def kernel(x_torso, x_joint, edge_index_tj, edge_index_jt, edge_index_jj,
           Wi_t, bi_t, Wi_j, bi_j,
           Wr1_tj, br1_tj, Wq1_tj, Wr1_jt, br1_jt, Wq1_jt, Wr1_jj, br1_jj, Wq1_jj,
           Wr2_tj, br2_tj, Wq2_tj, Wr2_jt, br2_jt, Wq2_jt, Wr2_jj, br2_jj, Wq2_jj,
           Wo_t, bo_t, Wo_j, bo_j):
    pad_src = (jnp.arange(E_PAD - E, dtype=jnp.int32) * 41) % N
    pad_dst = N + (jnp.arange(E_PAD - E, dtype=jnp.int32) & 7)

    def _prep(ei, q0):
        src = jnp.concatenate([ei[0], pad_src])
        dst = jnp.concatenate([ei[1], pad_dst])
        s4 = 4 * src + q0
        return (s4.reshape(NCH, CH), (s4 + 1).reshape(NCH, CH),
                dst.reshape(NCH, CH))

    sl_tj, sh_tj, d_tj = _prep(edge_index_tj, 0)
    sl_jt, sh_jt, d_jt = _prep(edge_index_jt, 2)
    sl_jj, sh_jj, d_jj = _prep(edge_index_jj, 0)
    r2 = lambda b: b.reshape(1, -1)
    view = lambda p: p.reshape(4 * N, HW)
    wq1_j = Wq1_tj + Wq1_jj
    bq1_j = r2(br1_tj + br1_jj)
    wq2_j = Wq2_tj + Wq2_jj
    bq2_j = r2(br2_tj + br2_jj)

    a1, b1, q1 = _f1(x_torso, x_joint, Wi_t, r2(bi_t), Wi_j, r2(bi_j),
                     Wr1_tj, Wr1_jt, Wr1_jj, Wq1_jt, r2(br1_jt), wq1_j, bq1_j)
    mt128, mj128 = _sc_segsum(
        view(a1), view(b1), q1,
        sl_jt, sh_jt, d_jt, sl_tj, sh_tj, d_tj, sl_jj, sh_jj, d_jj)
    a2, b2, q2 = _f2(mt128, mj128, Wr2_tj, Wr2_jt, Wr2_jj,
                     Wq2_jt, r2(br2_jt), wq2_j, bq2_j)
    mt128, mj128 = _sc_segsum(
        view(a2), view(b2), q2,
        sl_jt, sh_jt, d_jt, sl_tj, sh_tj, d_tj, sl_jj, sh_jj, d_jj)
    loc_t, scale_t, loc_j, scale_j = _f3(mt128, mj128, Wo_t, r2(bo_t),
                                         Wo_j, r2(bo_j))
    return (loc_t, scale_t, loc_j, scale_j)


# speedup vs baseline: 7.1356x; 1.0013x over previous
import math

import jax
import jax.numpy as jnp
from jax import lax
from jax.experimental import pallas as pl
from jax.experimental.pallas import tpu as pltpu
from jax.experimental.pallas import tpu_sc as plsc

N = 50000
E = 200000
D = 128
H = 64
HW = 32
O = 16
_BIAS = math.log(math.exp(1.0) - 1.0)

NC = 2
NS = 16
PT = 3128
PT_LAST = N - (NS - 1) * PT
ACC_ROWS = NS * PT
CH = 112
E_PAD = 200704
NCH = E_PAD // CH
CPT = NCH // NS
HSLAB = CPT // 2
RING = 4


def _sc_segsum_body(va, vb, q128,
                    sl_jt, sh_jt, d_jt, sl_tj, sh_tj, d_tj,
                    sl_jj, sh_jj, d_jj,
                    mt128, mj128,
                    acc, srcb, dstb, rows, gs, ss):
    c = lax.axis_index("c")
    s = lax.axis_index("s")
    c_lo = s * CPT

    def init_acc(col0):
        r0 = s * PT

        @pl.when(s < NS - 1)
        def _():
            pltpu.sync_copy(q128.at[pl.ds(r0, PT), pl.ds(col0, HW)],
                            acc.at[pl.ds(r0, PT)])

        @pl.when(s == NS - 1)
        def _():
            pltpu.sync_copy(q128.at[pl.ds(r0, PT_LAST), pl.ds(col0, HW)],
                            acc.at[pl.ds(r0, PT_LAST)])
            pltpu.sync_copy(q128.at[pl.ds(0, PT - PT_LAST), pl.ds(col0, HW)],
                            acc.at[pl.ds(r0 + PT_LAST, PT - PT_LAST)])

    def init_pass(base):
        @pl.when(c == 0)
        def _():
            init_acc(base)

        @pl.when(c == 1)
        def _():
            init_acc(base + HW)

    def accumulate(gsrc, s2d, d2d):
        def fire_g(k, p):
            pltpu.async_copy(gsrc.at[srcb.at[k]], rows.at[p], gs.at[p])

        def drain_g(k, p):
            pltpu.make_async_copy(gsrc.at[srcb.at[k]], rows.at[p], gs.at[p]).wait()

        def fire_s(k, p):
            pltpu.async_copy(rows.at[p], acc.at[dstb.at[k]], ss.at[p], add=True)

        def drain_s(k, p):
            pltpu.make_async_copy(rows.at[p], acc.at[dstb.at[k]], ss.at[p]).wait()

        def body(k4, carry):
            for p in range(RING):
                k = RING * k4 + p
                drain_g(k, p)
                fire_s(k, p)

                @pl.when(k + RING < HSLAB)
                def _():
                    drain_s(k, p)
                    fire_g(k + RING, p)

            return carry

        for h in range(CPT // HSLAB):
            pltpu.sync_copy(s2d.at[pl.ds(c_lo + h * HSLAB, HSLAB)], srcb)
            pltpu.sync_copy(d2d.at[pl.ds(c_lo + h * HSLAB, HSLAB)], dstb)
            for p in range(RING):
                fire_g(p, p)
            lax.fori_loop(0, HSLAB // RING, body, 0)
            for p in range(RING):
                drain_s(HSLAB - RING + p, p)

    def dump(out128):
        r0 = s * PT

        def to(col0, n):
            pltpu.sync_copy(acc.at[pl.ds(r0, n)],
                            out128.at[pl.ds(r0, n), pl.ds(col0, HW)])

        @pl.when((c == 0) & (s < NS - 1))
        def _():
            to(0, PT)

        @pl.when((c == 0) & (s == NS - 1))
        def _():
            to(0, PT_LAST)

        @pl.when((c == 1) & (s < NS - 1))
        def _():
            to(HW, PT)

        @pl.when((c == 1) & (s == NS - 1))
        def _():
            to(HW, PT_LAST)

    def conv_pass(gview, s_lo, s_hi, d2d):
        @pl.when(c == 0)
        def _():
            accumulate(gview, s_lo, d2d)

        @pl.when(c == 1)
        def _():
            accumulate(gview, s_hi, d2d)

    with jax.named_scope("sc_init1"):
        init_pass(0)
    plsc.subcore_barrier()
    with jax.named_scope("sc_conv_jt"):
        conv_pass(va, sl_jt, sh_jt, d_jt)
    plsc.subcore_barrier()
    with jax.named_scope("sc_dump1"):
        dump(mt128)
    with jax.named_scope("sc_init2"):
        init_pass(2 * HW)
    plsc.subcore_barrier()
    with jax.named_scope("sc_conv_tj"):
        conv_pass(va, sl_tj, sh_tj, d_tj)
    with jax.named_scope("sc_conv_jj"):
        conv_pass(vb, sl_jj, sh_jj, d_jj)
    plsc.subcore_barrier()
    with jax.named_scope("sc_dump2"):
        dump(mj128)


_sc_segsum = pl.kernel(
    _sc_segsum_body,
    out_type=tuple(jax.ShapeDtypeStruct((N, 4 * HW), jnp.float32)
                   for _ in range(2)),
    mesh=plsc.VectorSubcoreMesh(core_axis_name="c", subcore_axis_name="s"),
    scratch_types=(
        pltpu.VMEM_SHARED((ACC_ROWS, HW), jnp.float32),
        pltpu.VMEM((HSLAB, CH), jnp.int32),
        pltpu.VMEM((HSLAB, CH), jnp.int32),
        pltpu.VMEM((RING, CH, HW), jnp.float32),
        pltpu.SemaphoreType.DMA((RING,)),
        pltpu.SemaphoreType.DMA((RING,)),
    ),
    compiler_params=pltpu.CompilerParams(use_tc_tiling_on_sc=False),
)


R = 2000
_P = jax.lax.Precision.DEFAULT


def _dot(a, b):
    return jnp.dot(a, b, precision=_P, preferred_element_type=jnp.float32)


def _cat(a, b):
    return jnp.concatenate([a, b], axis=1)


def _f1_body(xt, xj, wit, bit, wij, bij, wr_tj, wr_jt, wr_jj,
             wq_t, bq_t, wq_j, bq_j, a_out, b_out, q_out):
    ht = _dot(xt[...], wit[...]) + bit[...]
    hj = _dot(xj[...], wij[...]) + bij[...]
    a_out[...] = _cat(_dot(ht, wr_tj[...]), _dot(hj, wr_jt[...]))
    b_out[...] = _cat(_dot(hj, wr_jj[...]),
                      jnp.zeros((R, H), jnp.float32))
    q_out[...] = _cat(_dot(ht, wq_t[...]) + bq_t[...],
                      _dot(hj, wq_j[...]) + bq_j[...])


def _f2_body(mt, mj, wr_tj, wr_jt, wr_jj, wq_t, bq_t, wq_j, bq_j,
             a_out, b_out, q_out):
    ht = jnp.tanh(mt[:, :H])
    hj = jnp.tanh(mj[:, :H])
    a_out[...] = _cat(_dot(ht, wr_tj[...]), _dot(hj, wr_jt[...]))
    b_out[...] = _cat(_dot(hj, wr_jj[...]),
                      jnp.zeros((R, H), jnp.float32))
    q_out[...] = _cat(_dot(ht, wq_t[...]) + bq_t[...],
                      _dot(hj, wq_j[...]) + bq_j[...])


def _f3_body(mt, mj, wot, bot, woj, boj,
             loc_t_out, scale_t_out, loc_j_out, scale_j_out):
    def head(m, wo, bo, loc_ref, scale_ref):
        h = jnp.tanh(m[:, :H])
        y = jnp.tanh(_dot(h, wo[...]) + bo[...])
        loc_ref[...] = y[:, :O // 2]
        v = y[:, O // 2:] + _BIAS
        sp = jnp.log1p(jnp.exp(-jnp.abs(v))) + jnp.maximum(v, 0.0)
        scale_ref[...] = jnp.maximum(sp, 1e-4)

    head(mt, wot, bot, loc_t_out, scale_t_out)
    head(mj, woj, boj, loc_j_out, scale_j_out)


def _spec(rows, cols):
    return pl.BlockSpec((rows, cols), lambda i: (i, 0))


def _w_spec(r, cc):
    return pl.BlockSpec((r, cc), lambda i: (0, 0))


def _call(body, in_rc, w_shapes, out_rc):
    grid = N // R
    in_specs = [_spec(*rc) for rc in in_rc] + [_w_spec(*sh) for sh in w_shapes]
    return pl.pallas_call(
        body,
        grid=(grid,),
        in_specs=in_specs,
        out_specs=[_spec(rc[0], rc[1]) for rc in out_rc],
        out_shape=[jax.ShapeDtypeStruct((N, rc[-1]), jnp.float32)
                   for rc in out_rc],
    )


_P128 = (R, 4 * HW)
_WH = (H, H)
_B = (1, H)
_f1 = _call(_f1_body, [(R, D), (R, D)],
            [(D, H), _B, (D, H), _B, _WH, _WH, _WH, _WH, _B, _WH, _B],
            [_P128, _P128, _P128])
_f2 = _call(_f2_body, [_P128, _P128],
            [_WH, _WH, _WH, _WH, _B, _WH, _B],
            [_P128, _P128, _P128])
_f3 = _call(_f3_body, [_P128, _P128],
            [(H, O), (1, O), (H, O), (1, O)],
            [(R, O // 2)] * 4)


def kernel(x_torso, x_joint, edge_index_tj, edge_index_jt, edge_index_jj,
           Wi_t, bi_t, Wi_j, bi_j,
           Wr1_tj, br1_tj, Wq1_tj, Wr1_jt, br1_jt, Wq1_jt, Wr1_jj, br1_jj, Wq1_jj,
           Wr2_tj, br2_tj, Wq2_tj, Wr2_jt, br2_jt, Wq2_jt, Wr2_jj, br2_jj, Wq2_jj,
           Wo_t, bo_t, Wo_j, bo_j):
    pad_src = (jnp.arange(E_PAD - E, dtype=jnp.int32) * 41) % N
    pad_dst = N + (jnp.arange(E_PAD - E, dtype=jnp.int32) & 7)

    def _prep(ei, q0):
        src = jnp.concatenate([ei[0], pad_src])
        dst = jnp.concatenate([ei[1], pad_dst])
        s4 = 4 * src + q0
        return (s4.reshape(NCH, CH), (s4 + 1).reshape(NCH, CH),
                dst.reshape(NCH, CH))

    sl_tj, sh_tj, d_tj = _prep(edge_index_tj, 0)
    sl_jt, sh_jt, d_jt = _prep(edge_index_jt, 2)
    sl_jj, sh_jj, d_jj = _prep(edge_index_jj, 0)
    r2 = lambda b: b.reshape(1, -1)
    view = lambda p: p.reshape(4 * N, HW)
    wq1_j = Wq1_tj + Wq1_jj
    bq1_j = r2(br1_tj + br1_jj)
    wq2_j = Wq2_tj + Wq2_jj
    bq2_j = r2(br2_tj + br2_jj)

    a1, b1, q1 = _f1(x_torso, x_joint, Wi_t, r2(bi_t), Wi_j, r2(bi_j),
                     Wr1_tj, Wr1_jt, Wr1_jj, Wq1_jt, r2(br1_jt), wq1_j, bq1_j)
    mt128, mj128 = _sc_segsum(
        view(a1), view(b1), q1,
        sl_jt, sh_jt, d_jt, sl_tj, sh_tj, d_tj, sl_jj, sh_jj, d_jj)
    a2, b2, q2 = _f2(mt128, mj128, Wr2_tj, Wr2_jt, Wr2_jj,
                     Wq2_jt, r2(br2_jt), wq2_j, bq2_j)
    mt128, mj128 = _sc_segsum(
        view(a2), view(b2), q2,
        sl_jt, sh_jt, d_jt, sl_tj, sh_tj, d_tj, sl_jj, sh_jj, d_jj)
    loc_t, scale_t, loc_j, scale_j = _f3(mt128, mj128, Wo_t, r2(bo_t),
                                         Wo_j, r2(bo_j))
    return (loc_t, scale_t, loc_j, scale_j)
